# Initial kernel scaffold; baseline (speedup 1.0000x reference)
import jax
import jax.numpy as jnp
from jax.experimental import pallas as pl


def kernel(trg_token_ids_batch):
    raise NotImplementedError("write your pallas kernel here")


# speedup vs baseline: 1.7386x; 1.7386x over previous
import functools

import jax
import jax.numpy as jnp
from jax.experimental import pallas as pl

SMOOTHING_VALUE = 0.1
PAD_TOKEN_ID = 0
TRG_VOCAB_SIZE = 100000
CONFIDENCE_VALUE = 1.0 - SMOOTHING_VALUE
SMOOTH = SMOOTHING_VALUE / (TRG_VOCAB_SIZE - 2)

ROW_BLOCK = 256
COL_BLOCK = 2048


def _fill_kernel(idx_ref, out_ref):
    j = pl.program_id(1)
    idx = idx_ref[:, :]
    cols = jax.lax.broadcasted_iota(jnp.int32, (ROW_BLOCK, COL_BLOCK), 1)
    cols = cols + j * COL_BLOCK
    is_target = cols == idx
    val = jnp.where(is_target, jnp.float32(CONFIDENCE_VALUE), jnp.float32(SMOOTH))
    val = jnp.where(idx == PAD_TOKEN_ID, jnp.float32(0.0), val)
    out_ref[:, :] = val


@jax.jit
def kernel(trg_token_ids_batch):
    b = trg_token_ids_batch.shape[0]
    idx = trg_token_ids_batch.astype(jnp.int32)
    grid = (b // ROW_BLOCK, pl.cdiv(TRG_VOCAB_SIZE, COL_BLOCK))
    return pl.pallas_call(
        _fill_kernel,
        grid=grid,
        in_specs=[pl.BlockSpec((ROW_BLOCK, 1), lambda i, j: (i, 0))],
        out_specs=pl.BlockSpec((ROW_BLOCK, COL_BLOCK), lambda i, j: (i, j)),
        out_shape=jax.ShapeDtypeStruct((b, TRG_VOCAB_SIZE), jnp.float32),
    )(idx)


# speedup vs baseline: 1.7797x; 1.0237x over previous
import functools

import jax
import jax.numpy as jnp
from jax.experimental import pallas as pl

SMOOTHING_VALUE = 0.1
PAD_TOKEN_ID = 0
TRG_VOCAB_SIZE = 100000
CONFIDENCE_VALUE = 1.0 - SMOOTHING_VALUE
SMOOTH = SMOOTHING_VALUE / (TRG_VOCAB_SIZE - 2)

ROW_BLOCK = 8
COL_BLOCK = TRG_VOCAB_SIZE


def _fill_kernel(idx_ref, out_ref):
    j = pl.program_id(1)
    idx = idx_ref[:, :]
    cols = jax.lax.broadcasted_iota(jnp.int32, (ROW_BLOCK, COL_BLOCK), 1)
    cols = cols + j * COL_BLOCK
    is_target = cols == idx
    val = jnp.where(is_target, jnp.float32(CONFIDENCE_VALUE), jnp.float32(SMOOTH))
    val = jnp.where(idx == PAD_TOKEN_ID, jnp.float32(0.0), val)
    out_ref[:, :] = val


@jax.jit
def kernel(trg_token_ids_batch):
    b = trg_token_ids_batch.shape[0]
    idx = trg_token_ids_batch.astype(jnp.int32)
    grid = (b // ROW_BLOCK, pl.cdiv(TRG_VOCAB_SIZE, COL_BLOCK))
    return pl.pallas_call(
        _fill_kernel,
        grid=grid,
        in_specs=[pl.BlockSpec((ROW_BLOCK, 1), lambda i, j: (i, 0))],
        out_specs=pl.BlockSpec((ROW_BLOCK, COL_BLOCK), lambda i, j: (i, j)),
        out_shape=jax.ShapeDtypeStruct((b, TRG_VOCAB_SIZE), jnp.float32),
    )(idx)


# speedup vs baseline: 1.8837x; 1.0584x over previous
import jax
import jax.numpy as jnp
from jax.experimental import pallas as pl
from jax.experimental.pallas import tpu as pltpu

SMOOTHING_VALUE = 0.1
PAD_TOKEN_ID = 0
TRG_VOCAB_SIZE = 100000
CONFIDENCE_VALUE = 1.0 - SMOOTHING_VALUE
SMOOTH = SMOOTHING_VALUE / (TRG_VOCAB_SIZE - 2)

ROWS = 8
NBUF = 4
BATCH = 1024
NSTEPS = BATCH // ROWS


def _fill_kernel(idx_ref, out_ref, buf, sem):
    g = pl.program_id(0)
    slot = jax.lax.rem(g, NBUF)

    @pl.when(g >= NBUF)
    def _():
        pltpu.make_async_copy(
            buf.at[slot], out_ref.at[pl.ds((g - NBUF) * ROWS, ROWS), :], sem.at[slot]
        ).wait()

    idx = idx_ref[pl.ds(g * ROWS, ROWS), :]
    cols = jax.lax.broadcasted_iota(jnp.int32, (ROWS, TRG_VOCAB_SIZE), 1)
    val = jnp.where(cols == idx, jnp.float32(CONFIDENCE_VALUE), jnp.float32(SMOOTH))
    val = jnp.where(idx == PAD_TOKEN_ID, jnp.float32(0.0), val)
    buf[slot] = val

    pltpu.make_async_copy(
        buf.at[slot], out_ref.at[pl.ds(g * ROWS, ROWS), :], sem.at[slot]
    ).start()

    @pl.when(g == NSTEPS - 1)
    def _():
        for k in range(NBUF):
            step = NSTEPS - NBUF + k
            s = step % NBUF
            pltpu.make_async_copy(
                buf.at[s], out_ref.at[pl.ds(step * ROWS, ROWS), :], sem.at[s]
            ).wait()


@jax.jit
def kernel(trg_token_ids_batch):
    idx = trg_token_ids_batch.astype(jnp.int32)
    return pl.pallas_call(
        _fill_kernel,
        grid=(NSTEPS,),
        in_specs=[pl.BlockSpec(memory_space=pltpu.MemorySpace.VMEM)],
        out_specs=pl.BlockSpec(memory_space=pltpu.MemorySpace.HBM),
        out_shape=jax.ShapeDtypeStruct((BATCH, TRG_VOCAB_SIZE), jnp.float32),
        scratch_shapes=[
            pltpu.VMEM((NBUF, ROWS, TRG_VOCAB_SIZE), jnp.float32),
            pltpu.SemaphoreType.DMA((NBUF,)),
        ],
    )(idx)
